# baseline (speedup 1.0000x reference)
import functools

import jax
import jax.numpy as jnp
from jax import lax
from jax.experimental import pallas as pl
from jax.experimental.pallas import tpu as pltpu
from jax.experimental.pallas import tpu_sc as plsc

NC = 2
NS = 16
NW = NC * NS
BATCH = 16384
BPW = BATCH // NW
NROW = 100000
CW = NROW // 8


@jax.jit
def _sc_gather_concat(intr, rot, trans, idx):
    tab8 = jnp.concatenate([
        intr.T.reshape(-1),
        rot.T.reshape(-1),
        trans.T.reshape(-1),
    ]).reshape(10 * NROW // 8, 8)
    mesh = plsc.VectorSubcoreMesh(core_axis_name="c", subcore_axis_name="s")

    @functools.partial(
        pl.kernel,
        mesh=mesh,
        out_type=jax.ShapeDtypeStruct((10, BATCH), jnp.float32),
        compiler_params=pltpu.CompilerParams(
            use_tc_tiling_on_sc=False, needs_layout_passes=False),
        scratch_types=[
            pltpu.VMEM((BPW,), jnp.int32),
            pltpu.VMEM((BPW,), jnp.int32),
            pltpu.VMEM((10 * BPW, 8), jnp.float32),
            pltpu.VMEM((10, BPW), jnp.float32),
            pltpu.SemaphoreType.DMA,
        ],
    )
    def k(tab_hbm, idx_hbm, out_hbm, idx_v, gl, vbuf, out_v, sem):
        wid = lax.axis_index("s") * NC + lax.axis_index("c")
        base = wid * BPW
        pltpu.sync_copy(idx_hbm.at[pl.ds(base, BPW)], idx_v)

        iota = lax.iota(jnp.int32, 16)

        def prep(ch, carry):
            iv = idx_v[pl.ds(ch * 16, 16)]
            gl[pl.ds(ch * 16, 16)] = lax.shift_right_logical(iv, 3)
            return carry

        lax.fori_loop(0, BPW // 16, prep, 0)

        copies = []
        for q in range(10):
            copies.append(pltpu.async_copy(
                tab_hbm.at[pl.ds(q * CW, CW), :].at[gl],
                vbuf.at[pl.ds(q * BPW, BPW), :], sem))
        for cpy in copies:
            cpy.wait()

        def body(chunk, carry):
            rows = chunk * 16 + iota
            iv = idx_v[pl.ds(chunk * 16, 16)]
            off = iv & 7
            for q in range(10):
                vals = plsc.load_gather(vbuf, [q * BPW + rows, off])
                plsc.store_scatter(out_v, [jnp.full((16,), q, jnp.int32), rows],
                                   vals)
            return carry

        lax.fori_loop(0, BPW // 16, body, 0)
        pltpu.sync_copy(out_v, out_hbm.at[:, pl.ds(base, BPW)])

    return k(tab8, idx).T


def kernel(intrinsic_deltas, rotation_deltas, translation_deltas, camera_idx):
    return _sc_gather_concat(intrinsic_deltas, rotation_deltas,
                             translation_deltas,
                             camera_idx.astype(jnp.int32))


# speedup vs baseline: 1.5988x; 1.5988x over previous
import functools

import jax
import jax.numpy as jnp
from jax import lax
from jax.experimental import pallas as pl
from jax.experimental.pallas import tpu as pltpu
from jax.experimental.pallas import tpu_sc as plsc

NC = 2
NS = 16
NW = NC * NS
BATCH = 16384
BPW = BATCH // NW
NROW = 100000
CW = NROW // 8


@jax.jit
def _sc_gather_concat(intr, rot, trans, idx):
    intr8 = intr.T.reshape(4 * NROW // 8, 8)
    rot8 = rot.T.reshape(3 * NROW // 8, 8)
    trans8 = trans.T.reshape(3 * NROW // 8, 8)
    mesh = plsc.VectorSubcoreMesh(core_axis_name="c", subcore_axis_name="s")

    @functools.partial(
        pl.kernel,
        mesh=mesh,
        out_type=jax.ShapeDtypeStruct((10, BATCH), jnp.float32),
        compiler_params=pltpu.CompilerParams(
            use_tc_tiling_on_sc=False, needs_layout_passes=False),
        scratch_types=[
            pltpu.VMEM((BPW,), jnp.int32),
            pltpu.VMEM((BPW,), jnp.int32),
            pltpu.VMEM((10 * BPW, 8), jnp.float32),
            pltpu.VMEM((10, BPW), jnp.float32),
            pltpu.SemaphoreType.DMA,
        ],
    )
    def k(intr_hbm, rot_hbm, trans_hbm, idx_hbm, out_hbm,
          idx_v, gl, vbuf, out_v, sem):
        wid = lax.axis_index("s") * NC + lax.axis_index("c")
        base = wid * BPW
        pltpu.sync_copy(idx_hbm.at[pl.ds(base, BPW)], idx_v)

        iota = lax.iota(jnp.int32, 16)

        def prep(ch, carry):
            iv = idx_v[pl.ds(ch * 16, 16)]
            gl[pl.ds(ch * 16, 16)] = lax.shift_right_logical(iv, 3)
            return carry

        lax.fori_loop(0, BPW // 16, prep, 0)

        sources = ([intr_hbm.at[pl.ds(c * CW, CW), :] for c in range(4)]
                   + [rot_hbm.at[pl.ds(c * CW, CW), :] for c in range(3)]
                   + [trans_hbm.at[pl.ds(c * CW, CW), :] for c in range(3)])
        copies = []
        for q, src in enumerate(sources):
            copies.append(pltpu.async_copy(
                src.at[gl], vbuf.at[pl.ds(q * BPW, BPW), :], sem))
        for cpy in copies:
            cpy.wait()

        def body(chunk, carry):
            rows = chunk * 16 + iota
            iv = idx_v[pl.ds(chunk * 16, 16)]
            off = iv & 7
            for q in range(10):
                vals = plsc.load_gather(vbuf, [q * BPW + rows, off])
                plsc.store_scatter(out_v, [jnp.full((16,), q, jnp.int32), rows],
                                   vals)
            return carry

        lax.fori_loop(0, BPW // 16, body, 0)
        pltpu.sync_copy(out_v, out_hbm.at[:, pl.ds(base, BPW)])

    return k(intr8, rot8, trans8, idx).T


def kernel(intrinsic_deltas, rotation_deltas, translation_deltas, camera_idx):
    return _sc_gather_concat(intrinsic_deltas, rotation_deltas,
                             translation_deltas,
                             camera_idx.astype(jnp.int32))


# speedup vs baseline: 1.6014x; 1.0017x over previous
import functools

import jax
import jax.numpy as jnp
from jax import lax
from jax.experimental import pallas as pl
from jax.experimental.pallas import tpu as pltpu
from jax.experimental.pallas import tpu_sc as plsc

NC = 2
NS = 16
NW = NC * NS
BATCH = 16384
BPW = BATCH // NW
NROW = 100000
CW = NROW // 8


@jax.jit
def _sc_gather_concat(intr, rot, trans, idx):
    intr8 = intr.T.reshape(4 * NROW // 8, 8)
    rot8 = rot.T.reshape(3 * NROW // 8, 8)
    trans8 = trans.T.reshape(3 * NROW // 8, 8)
    mesh = plsc.VectorSubcoreMesh(core_axis_name="c", subcore_axis_name="s")

    @functools.partial(
        pl.kernel,
        mesh=mesh,
        out_type=jax.ShapeDtypeStruct((10, BATCH), jnp.float32),
        compiler_params=pltpu.CompilerParams(
            use_tc_tiling_on_sc=False, needs_layout_passes=False,
            disable_bounds_checks=True, disable_semaphore_checks=True),
        scratch_types=[
            pltpu.VMEM((BPW,), jnp.int32),
            pltpu.VMEM((BPW,), jnp.int32),
            pltpu.VMEM((10 * BPW, 8), jnp.float32),
            pltpu.VMEM((10, BPW), jnp.float32),
            pltpu.SemaphoreType.DMA,
        ],
    )
    def k(intr_hbm, rot_hbm, trans_hbm, idx_hbm, out_hbm,
          idx_v, gl, vbuf, out_v, sem):
        wid = lax.axis_index("s") * NC + lax.axis_index("c")
        base = wid * BPW
        pltpu.sync_copy(idx_hbm.at[pl.ds(base, BPW)], idx_v)

        iota = lax.iota(jnp.int32, 16)

        def prep(ch, carry):
            iv = idx_v[pl.ds(ch * 16, 16)]
            gl[pl.ds(ch * 16, 16)] = lax.shift_right_logical(iv, 3)
            return carry

        lax.fori_loop(0, BPW // 16, prep, 0)

        sources = ([intr_hbm.at[pl.ds(c * CW, CW), :] for c in range(4)]
                   + [rot_hbm.at[pl.ds(c * CW, CW), :] for c in range(3)]
                   + [trans_hbm.at[pl.ds(c * CW, CW), :] for c in range(3)])
        copies = []
        for q, src in enumerate(sources):
            copies.append(pltpu.async_copy(
                src.at[gl], vbuf.at[pl.ds(q * BPW, BPW), :], sem))
        for cpy in copies:
            cpy.wait()

        def body(chunk, carry):
            rows = chunk * 16 + iota
            iv = idx_v[pl.ds(chunk * 16, 16)]
            off = iv & 7
            for q in range(10):
                vals = plsc.load_gather(vbuf, [q * BPW + rows, off])
                plsc.store_scatter(out_v, [jnp.full((16,), q, jnp.int32), rows],
                                   vals)
            return carry

        lax.fori_loop(0, BPW // 16, body, 0)
        pltpu.sync_copy(out_v, out_hbm.at[:, pl.ds(base, BPW)])

    return k(intr8, rot8, trans8, idx).T


def kernel(intrinsic_deltas, rotation_deltas, translation_deltas, camera_idx):
    return _sc_gather_concat(intrinsic_deltas, rotation_deltas,
                             translation_deltas,
                             camera_idx.astype(jnp.int32))


# speedup vs baseline: 1.6106x; 1.0058x over previous
import functools

import jax
import jax.numpy as jnp
from jax import lax
from jax.experimental import pallas as pl
from jax.experimental.pallas import tpu as pltpu
from jax.experimental.pallas import tpu_sc as plsc

NC = 2
NS = 16
NW = NC * NS
BATCH = 16384
BPW = BATCH // NW
NROW = 100000
CW = NROW // 8


@jax.jit
def _sc_gather_concat(intr, rot, trans, idx):
    intr8 = intr.T.reshape(4 * NROW // 8, 8)
    rot8 = rot.T.reshape(3 * NROW // 8, 8)
    trans8 = trans.T.reshape(3 * NROW // 8, 8)
    mesh = plsc.VectorSubcoreMesh(core_axis_name="c", subcore_axis_name="s")

    @functools.partial(
        pl.kernel,
        mesh=mesh,
        out_type=jax.ShapeDtypeStruct((10, BATCH), jnp.float32),
        compiler_params=pltpu.CompilerParams(
            use_tc_tiling_on_sc=False, needs_layout_passes=False,
            disable_bounds_checks=True, disable_semaphore_checks=True),
        scratch_types=[
            pltpu.VMEM((BPW,), jnp.int32),
            pltpu.VMEM((BPW,), jnp.int32),
            pltpu.VMEM((BPW,), jnp.int32),
            pltpu.VMEM((10 * BPW, 8), jnp.float32),
            pltpu.VMEM((10, BPW), jnp.float32),
            pltpu.SemaphoreType.DMA((10,)),
            pltpu.SemaphoreType.DMA,
        ],
    )
    def k(intr_hbm, rot_hbm, trans_hbm, idx_hbm, out_hbm,
          idx_v, gl, ofs, vbuf, out_v, sems, osem):
        wid = lax.axis_index("s") * NC + lax.axis_index("c")
        base = wid * BPW
        pltpu.sync_copy(idx_hbm.at[pl.ds(base, BPW)], idx_v)

        iota = lax.iota(jnp.int32, 16)

        def prep(ch, carry):
            iv = idx_v[pl.ds(ch * 16, 16)]
            gl[pl.ds(ch * 16, 16)] = lax.shift_right_logical(iv, 3)
            ofs[pl.ds(ch * 16, 16)] = iv & 7
            return carry

        lax.fori_loop(0, BPW // 16, prep, 0)

        sources = ([intr_hbm.at[pl.ds(c * CW, CW), :] for c in range(4)]
                   + [rot_hbm.at[pl.ds(c * CW, CW), :] for c in range(3)]
                   + [trans_hbm.at[pl.ds(c * CW, CW), :] for c in range(3)])
        copies = []
        for q, src in enumerate(sources):
            copies.append(pltpu.async_copy(
                src.at[gl], vbuf.at[pl.ds(q * BPW, BPW), :], sems.at[q]))

        out_copies = []
        for q in range(10):
            copies[q].wait()

            def body(chunk, carry, q=q):
                rows = chunk * 16 + iota
                off = ofs[pl.ds(chunk * 16, 16)]
                vals = plsc.load_gather(vbuf, [q * BPW + rows, off])
                plsc.store_scatter(out_v, [jnp.full((16,), q, jnp.int32), rows],
                                   vals)
                return carry

            lax.fori_loop(0, BPW // 16, body, 0)
            out_copies.append(pltpu.async_copy(
                out_v.at[q], out_hbm.at[q, pl.ds(base, BPW)], osem))
        for cpy in out_copies:
            cpy.wait()

    return k(intr8, rot8, trans8, idx).T


def kernel(intrinsic_deltas, rotation_deltas, translation_deltas, camera_idx):
    return _sc_gather_concat(intrinsic_deltas, rotation_deltas,
                             translation_deltas,
                             camera_idx.astype(jnp.int32))
